# baseline (speedup 1.0000x reference)
import jax
import jax.numpy as jnp
from jax.experimental import pallas as pl
from jax.experimental.pallas import tpu as pltpu

_N = 10000
_H = 128
_BM = 400
_RB = 1
_R = _RB * _BM


def _body(adj_ref, res_ref, x_ref, W1_ref, b1_ref, W2_ref, b2_ref, Wt1_ref,
          bt1_ref, Wt2_ref, bt2_ref, Wp_ref, bp_ref, rep_ref, tau_ref, e_ref,
          s_ref, h1_ref, acc_ref):
    p = pl.program_id(0)
    i = pl.program_id(1)

    @pl.when((p == 0) & (i == 0))
    def _():
        s_ref[...] = jnp.dot(x_ref[...].astype(jnp.float32), W1_ref[...],
                             preferred_element_type=jnp.float32)

    @pl.when((p == 1) & (i == 0))
    def _():
        s_ref[...] = jnp.dot(h1_ref[...].astype(jnp.float32), W2_ref[...],
                             preferred_element_type=jnp.float32)

    @pl.when(i < _RB)
    def _():
        acc_ref[...] = jnp.dot(res_ref[pl.ds(i * _BM, _BM), :], s_ref[...],
                               preferred_element_type=jnp.float32)

    @pl.when(i >= _RB)
    def _():
        acc_ref[...] = jnp.dot(adj_ref[...], s_ref[...],
                               preferred_element_type=jnp.float32)

    @pl.when(p == 0)
    def _():
        h1_ref[pl.ds(i * _BM, _BM), :] = jnp.maximum(
            acc_ref[...] + b1_ref[...], 0.0).astype(jnp.bfloat16)

    @pl.when(p == 1)
    def _():
        h2 = jnp.maximum(acc_ref[...] + b2_ref[...], 0.0)
        rep_ref[...] = h2
        t = jnp.maximum(
            jnp.dot(h2, Wt1_ref[...], preferred_element_type=jnp.float32)
            + bt1_ref[...], 0.0)
        tau_ref[...] = (jnp.dot(t, Wt2_ref[...],
                                preferred_element_type=jnp.float32)
                        + bt2_ref[...])
        e_ref[...] = jax.nn.sigmoid(
            jnp.dot(h2, Wp_ref[...], preferred_element_type=jnp.float32)
            + bp_ref[...])


def kernel(x, adj, W1, b1, W2, b2, Wt1, bt1, Wt2, bt2, Wp, bp):
    full = lambda p, i: (0, 0)
    stream_idx = lambda p, i: (jnp.maximum(i, _RB), 0)
    out_idx = lambda p, i: (jnp.where(p == 0, 0, i), 0)
    rep, tau, e = pl.pallas_call(
        _body,
        grid=(2, _N // _BM),
        in_specs=[
            pl.BlockSpec((_BM, _N), stream_idx),
            pl.BlockSpec((_R, _N), full),
            pl.BlockSpec((_N, _H), full),
            pl.BlockSpec((_H, _H), full),
            pl.BlockSpec((1, _H), full),
            pl.BlockSpec((_H, _H), full),
            pl.BlockSpec((1, _H), full),
            pl.BlockSpec((_H, _H), full),
            pl.BlockSpec((1, _H), full),
            pl.BlockSpec((_H, 1), full),
            pl.BlockSpec((1, 1), full),
            pl.BlockSpec((_H, 1), full),
            pl.BlockSpec((1, 1), full),
        ],
        out_specs=[
            pl.BlockSpec((_BM, _H), out_idx),
            pl.BlockSpec((_BM, 1), out_idx),
            pl.BlockSpec((_BM, 1), out_idx),
        ],
        out_shape=[
            jax.ShapeDtypeStruct((_N, _H), jnp.float32),
            jax.ShapeDtypeStruct((_N, 1), jnp.float32),
            jax.ShapeDtypeStruct((_N, 1), jnp.float32),
        ],
        scratch_shapes=[
            pltpu.VMEM((_N, _H), jnp.float32),
            pltpu.VMEM((_N, _H), jnp.bfloat16),
            pltpu.VMEM((_BM, _H), jnp.float32),
        ],
        compiler_params=pltpu.CompilerParams(
            dimension_semantics=("arbitrary", "arbitrary")),
    )(adj, adj, x.astype(jnp.bfloat16), W1, b1.reshape(1, _H), W2,
      b2.reshape(1, _H), Wt1,
      bt1.reshape(1, _H), Wt2, bt2.reshape(1, 1), Wp, bp.reshape(1, 1))
    tau = tau[:, 0]
    e = e[:, 0]
    z = jnp.zeros_like(tau)
    return (e, z, tau, tau, tau, z, z, rep)


# speedup vs baseline: 1.0161x; 1.0161x over previous
import jax
import jax.numpy as jnp
from jax.experimental import pallas as pl
from jax.experimental.pallas import tpu as pltpu

_N = 10000
_H = 128
_BM = 400


def _body(adj_ref, x_ref, W1_ref, b1_ref, W2_ref, b2_ref, Wt1_ref, bt1_ref,
          Wt2_ref, bt2_ref, Wp_ref, bp_ref, rep_ref, tau_ref, e_ref,
          s_ref, h1_ref):
    p = pl.program_id(0)
    i = pl.program_id(1)

    @pl.when((p == 0) & (i == 0))
    def _():
        s_ref[...] = jnp.dot(x_ref[...], W1_ref[...],
                             preferred_element_type=jnp.float32)

    @pl.when((p == 1) & (i == 0))
    def _():
        s_ref[...] = jnp.dot(h1_ref[...], W2_ref[...],
                             preferred_element_type=jnp.float32)

    acc = jnp.dot(adj_ref[...], s_ref[...],
                  preferred_element_type=jnp.float32)

    @pl.when(p == 0)
    def _():
        h1_ref[pl.ds(i * _BM, _BM), :] = jnp.maximum(acc + b1_ref[...], 0.0)

    @pl.when(p == 1)
    def _():
        h2 = jnp.maximum(acc + b2_ref[...], 0.0)
        rep_ref[...] = h2
        t = jnp.maximum(
            jnp.dot(h2, Wt1_ref[...], preferred_element_type=jnp.float32)
            + bt1_ref[...], 0.0)
        tau_ref[...] = (jnp.dot(t, Wt2_ref[...],
                                preferred_element_type=jnp.float32)
                        + bt2_ref[...])
        e_ref[...] = jax.nn.sigmoid(
            jnp.dot(h2, Wp_ref[...], preferred_element_type=jnp.float32)
            + bp_ref[...])


def kernel(x, adj, W1, b1, W2, b2, Wt1, bt1, Wt2, bt2, Wp, bp):
    full = lambda p, i: (0, 0)
    out_idx = lambda p, i: (i * p, 0)
    rep, tau, e = pl.pallas_call(
        _body,
        grid=(2, _N // _BM),
        in_specs=[
            pl.BlockSpec((_BM, _N), lambda p, i: (i, 0)),
            pl.BlockSpec((_N, _H), full),
            pl.BlockSpec((_H, _H), full),
            pl.BlockSpec((1, _H), full),
            pl.BlockSpec((_H, _H), full),
            pl.BlockSpec((1, _H), full),
            pl.BlockSpec((_H, _H), full),
            pl.BlockSpec((1, _H), full),
            pl.BlockSpec((_H, 1), full),
            pl.BlockSpec((1, 1), full),
            pl.BlockSpec((_H, 1), full),
            pl.BlockSpec((1, 1), full),
        ],
        out_specs=[
            pl.BlockSpec((_BM, _H), out_idx),
            pl.BlockSpec((_BM, 1), out_idx),
            pl.BlockSpec((_BM, 1), out_idx),
        ],
        out_shape=[
            jax.ShapeDtypeStruct((_N, _H), jnp.float32),
            jax.ShapeDtypeStruct((_N, 1), jnp.float32),
            jax.ShapeDtypeStruct((_N, 1), jnp.float32),
        ],
        scratch_shapes=[
            pltpu.VMEM((_N, _H), jnp.float32),
            pltpu.VMEM((_N, _H), jnp.float32),
        ],
        compiler_params=pltpu.CompilerParams(
            dimension_semantics=("arbitrary", "arbitrary")),
    )(adj, x, W1, b1.reshape(1, _H), W2, b2.reshape(1, _H), Wt1,
      bt1.reshape(1, _H), Wt2, bt2.reshape(1, 1), Wp, bp.reshape(1, 1))
    tau = tau[:, 0]
    e = e[:, 0]
    z = jnp.zeros_like(tau)
    return (e, z, tau, tau, tau, z, z, rep)


# speedup vs baseline: 1.0247x; 1.0085x over previous
import jax
import jax.numpy as jnp
from jax.experimental import pallas as pl
from jax.experimental.pallas import tpu as pltpu

_N = 10000
_H = 128
_BM = 400
_NB = _N // _BM


def _body(adj_ref, x_ref, W1_ref, b1_ref, W2_ref, b2_ref, Wt1_ref, bt1_ref,
          Wt2_ref, bt2_ref, Wp_ref, bp_ref, rep_ref, tau_ref, e_ref,
          s_ref, h1_ref):
    p = pl.program_id(0)
    i = pl.program_id(1)

    @pl.when((p == 0) & (i == 0))
    def _():
        s_ref[...] = jnp.dot(x_ref[...], W1_ref[...],
                             preferred_element_type=jnp.float32)

    @pl.when((p == 1) & (i == 0))
    def _():
        s_ref[...] = jnp.dot(h1_ref[...], W2_ref[...],
                             preferred_element_type=jnp.float32)

    acc = jnp.dot(adj_ref[...], s_ref[...],
                  preferred_element_type=jnp.float32)

    @pl.when(p == 0)
    def _():
        h1_ref[pl.ds(i * _BM, _BM), :] = jnp.maximum(acc + b1_ref[...], 0.0)


    @pl.when(p == 1)
    def _():
        h2 = jnp.maximum(acc + b2_ref[...], 0.0)
        rep_ref[...] = h2
        t = jnp.maximum(
            jnp.dot(h2, Wt1_ref[...], preferred_element_type=jnp.float32)
            + bt1_ref[...], 0.0)
        tau_ref[...] = (jnp.dot(t, Wt2_ref[...],
                                preferred_element_type=jnp.float32)
                        + bt2_ref[...])
        e_ref[...] = jax.nn.sigmoid(
            jnp.dot(h2, Wp_ref[...], preferred_element_type=jnp.float32)
            + bp_ref[...])


def kernel(x, adj, W1, b1, W2, b2, Wt1, bt1, Wt2, bt2, Wp, bp):
    full = lambda p, i: (0, 0)
    adj_idx = lambda p, i: (jnp.where(p == 0, i, _NB - 1 - i), 0)
    out_idx = lambda p, i: (_NB - 1 - i * p, 0)
    rep, tau, e = pl.pallas_call(
        _body,
        grid=(2, _N // _BM),
        in_specs=[
            pl.BlockSpec((_BM, _N), adj_idx),
            pl.BlockSpec((_N, _H), full),
            pl.BlockSpec((_H, _H), full),
            pl.BlockSpec((1, _H), full),
            pl.BlockSpec((_H, _H), full),
            pl.BlockSpec((1, _H), full),
            pl.BlockSpec((_H, _H), full),
            pl.BlockSpec((1, _H), full),
            pl.BlockSpec((_H, 1), full),
            pl.BlockSpec((1, 1), full),
            pl.BlockSpec((_H, 1), full),
            pl.BlockSpec((1, 1), full),
        ],
        out_specs=[
            pl.BlockSpec((_BM, _H), out_idx),
            pl.BlockSpec((_BM, 1), out_idx),
            pl.BlockSpec((_BM, 1), out_idx),
        ],
        out_shape=[
            jax.ShapeDtypeStruct((_N, _H), jnp.float32),
            jax.ShapeDtypeStruct((_N, 1), jnp.float32),
            jax.ShapeDtypeStruct((_N, 1), jnp.float32),
        ],
        scratch_shapes=[
            pltpu.VMEM((_N, _H), jnp.float32),
            pltpu.VMEM((_N, _H), jnp.float32),
        ],
        compiler_params=pltpu.CompilerParams(
            dimension_semantics=("arbitrary", "arbitrary")),
    )(adj, x, W1, b1.reshape(1, _H), W2, b2.reshape(1, _H), Wt1,
      bt1.reshape(1, _H), Wt2, bt2.reshape(1, 1), Wp, bp.reshape(1, 1))
    tau = tau[:, 0]
    e = e[:, 0]
    z = jnp.zeros_like(tau)
    return (e, z, tau, tau, tau, z, z, rep)


# speedup vs baseline: 1.0281x; 1.0032x over previous
import jax
import jax.numpy as jnp
from jax.experimental import pallas as pl
from jax.experimental.pallas import tpu as pltpu

_N = 10000
_H = 128
_BM = 400
_NB = _N // _BM
_T = 2 * _NB


def _copy(adj_ref, abuf_ref, sem_ref, blk, slot):
    return pltpu.make_async_copy(
        adj_ref.at[pl.ds(blk * _BM, _BM), :],
        abuf_ref.at[slot],
        sem_ref.at[slot])


def _body(adj_ref, x_ref, W1_ref, b1_ref, W2_ref, b2_ref, Wt1_ref, bt1_ref,
          Wt2_ref, bt2_ref, Wp_ref, bp_ref, rep_ref, tau_ref, e_ref,
          abuf_ref, sem_ref, s_ref, h1_ref):
    t = pl.program_id(0)

    @pl.when(t == 0)
    def _():
        _copy(adj_ref, abuf_ref, sem_ref, jnp.int32(0), jnp.int32(0)).start()
        _copy(adj_ref, abuf_ref, sem_ref, jnp.int32(1), jnp.int32(1)).start()
        s_ref[...] = jnp.dot(x_ref[...], W1_ref[...],
                             preferred_element_type=jnp.float32)

    @pl.when((t >= 1) & (t <= _NB - 2))
    def _():
        _copy(adj_ref, abuf_ref, sem_ref, t + 1, (t + 1) % 2).start()

    @pl.when((t >= _NB + 1) & (t <= _T - 2))
    def _():
        _copy(adj_ref, abuf_ref, sem_ref, 2 * _NB - 2 - t, t % 2).start()

    @pl.when(t == _NB)
    def _():
        s_ref[...] = jnp.dot(h1_ref[...], W2_ref[...],
                             preferred_element_type=jnp.float32)

    cs = jnp.where(t < _NB, t % 2, (t + 1) % 2)

    @pl.when((t != _NB) & (t != _NB + 1))
    def _():
        pltpu.make_async_copy(adj_ref.at[pl.ds(0, _BM), :],
                              abuf_ref.at[cs], sem_ref.at[cs]).wait()

    acc = jnp.dot(abuf_ref[cs], s_ref[...],
                  preferred_element_type=jnp.float32)

    @pl.when(t < _NB)
    def _():
        h1_ref[pl.ds(t * _BM, _BM), :] = jnp.maximum(acc + b1_ref[...], 0.0)

    @pl.when(t >= _NB)
    def _():
        h2 = jnp.maximum(acc + b2_ref[...], 0.0)
        rep_ref[...] = h2
        tt = jnp.maximum(
            jnp.dot(h2, Wt1_ref[...], preferred_element_type=jnp.float32)
            + bt1_ref[...], 0.0)
        tau_ref[...] = (jnp.dot(tt, Wt2_ref[...],
                                preferred_element_type=jnp.float32)
                        + bt2_ref[...])
        e_ref[...] = jax.nn.sigmoid(
            jnp.dot(h2, Wp_ref[...], preferred_element_type=jnp.float32)
            + bp_ref[...])


def kernel(x, adj, W1, b1, W2, b2, Wt1, bt1, Wt2, bt2, Wp, bp):
    full = lambda t: (0, 0)
    out_idx = lambda t: (jnp.where(t < _NB, _NB - 1, 2 * _NB - 1 - t), 0)
    rep, tau, e = pl.pallas_call(
        _body,
        grid=(_T,),
        in_specs=[
            pl.BlockSpec(memory_space=pltpu.MemorySpace.HBM),
            pl.BlockSpec((_N, _H), full),
            pl.BlockSpec((_H, _H), full),
            pl.BlockSpec((1, _H), full),
            pl.BlockSpec((_H, _H), full),
            pl.BlockSpec((1, _H), full),
            pl.BlockSpec((_H, _H), full),
            pl.BlockSpec((1, _H), full),
            pl.BlockSpec((_H, 1), full),
            pl.BlockSpec((1, 1), full),
            pl.BlockSpec((_H, 1), full),
            pl.BlockSpec((1, 1), full),
        ],
        out_specs=[
            pl.BlockSpec((_BM, _H), out_idx),
            pl.BlockSpec((_BM, 1), out_idx),
            pl.BlockSpec((_BM, 1), out_idx),
        ],
        out_shape=[
            jax.ShapeDtypeStruct((_N, _H), jnp.float32),
            jax.ShapeDtypeStruct((_N, 1), jnp.float32),
            jax.ShapeDtypeStruct((_N, 1), jnp.float32),
        ],
        scratch_shapes=[
            pltpu.VMEM((2, _BM, _N), jnp.float32),
            pltpu.SemaphoreType.DMA((2,)),
            pltpu.VMEM((_N, _H), jnp.float32),
            pltpu.VMEM((_N, _H), jnp.float32),
        ],
        compiler_params=pltpu.CompilerParams(
            dimension_semantics=("arbitrary",)),
    )(adj, x, W1, b1.reshape(1, _H), W2, b2.reshape(1, _H), Wt1,
      bt1.reshape(1, _H), Wt2, bt2.reshape(1, 1), Wp, bp.reshape(1, 1))
    tau = tau[:, 0]
    e = e[:, 0]
    z = jnp.zeros_like(tau)
    return (e, z, tau, tau, tau, z, z, rep)
